# baseline (speedup 1.0000x reference)
import functools

import jax
import jax.numpy as jnp
from jax import lax
from jax.experimental import pallas as pl
from jax.experimental.pallas import tpu as pltpu
from jax.experimental.pallas import tpu_sc as plsc

D = 256
KEV = 100000
PB = 4096
NBLK = -(-KEV // PB)
PADTOT = NBLK * PB
NEG = float("-inf")

NW = 32
SLICE = PADTOT // NW
NCH = SLICE // 16


def _score_body(claim_ref, Wq_ref, bq_ref, Wk_ref, bkc_ref,
                keys_ref, out_ref):
    c0 = claim_ref[0:1, :]
    qr = lax.dot_general(c0, Wq_ref[...], (((1,), (1,)), ((), ())),
                         preferred_element_type=jnp.float32)
    qr = qr + bq_ref[...][None, :]
    qn = qr / jnp.maximum(jnp.sqrt(jnp.sum(qr * qr)), 1e-12)
    zT = lax.dot_general(Wk_ref[...], keys_ref[...], (((1,), (1,)), ((), ())),
                         preferred_element_type=jnp.float32)
    zT = zT + bkc_ref[...]
    num = lax.dot_general(qn, zT, (((1,), (0,)), ((), ())),
                          preferred_element_type=jnp.float32)
    d2 = jnp.sum(zT * zT, axis=0, keepdims=True)
    s = num / jnp.maximum(jnp.sqrt(d2), 1e-12)
    gid = (lax.broadcasted_iota(jnp.int32, (1, PB), 1)
           + pl.program_id(0) * PB)
    out_ref[...] = jnp.where(gid < KEV, s, NEG).reshape(PB)


def _merge16(bk, bi, ck, ci):
    ck, ci = plsc.sort_key_val(ck, ci, descending=True)
    bk, bi = plsc.sort_key_val(bk, bi)
    take = ck >= bk
    return jnp.where(take, ck, bk), jnp.where(take, ci, bi)


def _sc_select(scores_hbm, ck_out, ci_out, chunk_v, stage_k, stage_i):
    wid = lax.axis_index("c") * 16 + lax.axis_index("s")
    pltpu.sync_copy(scores_hbm.at[pl.ds(wid * SLICE, SLICE)], chunk_v)
    lanes = lax.iota(jnp.int32, 16)

    def body(j, carry):
        bk, bi = carry
        v = chunk_v[pl.ds(j * 16, 16)]
        gi = wid * SLICE + j * 16 + lanes
        return _merge16(bk, bi, v, gi)

    bk0 = jnp.full((16,), NEG, jnp.float32)
    bi0 = jnp.zeros((16,), jnp.int32)
    bk, bi = lax.fori_loop(0, NCH, body, (bk0, bi0))
    stage_k[...] = bk
    stage_i[...] = bi
    pltpu.sync_copy(stage_k, ck_out.at[wid])
    pltpu.sync_copy(stage_i, ci_out.at[wid])


def _select_mlp_body(ck_ref, ci_ref, values_ref, claim_ref, V1_ref, c1_ref,
                     V2_ref, c2_ref, retr_ref, ts_ref, cons_ref, conf_ref,
                     k_ref, sem):
    k_ref[...] = ck_ref[...]
    for i in range(5):
        k = k_ref[...]
        m = jnp.max(k)
        idx = jnp.min(jnp.where(k == m, ci_ref[...], KEV))
        ts_ref[i] = m
        cp = pltpu.make_async_copy(values_ref.at[pl.ds(idx, 1), :],
                                   retr_ref.at[pl.ds(i, 1), :], sem)
        cp.start()
        cp.wait()
        k_ref[...] = jnp.where(ci_ref[...] == idx, NEG, k)
    ev = retr_ref[...]
    A = V1_ref[:, 0:D]
    Bm = V1_ref[:, D:2 * D]
    ca = lax.dot_general(claim_ref[...], A, (((1,), (1,)), ((), ())),
                         preferred_element_type=jnp.float32)
    eb = lax.dot_general(ev, Bm, (((1,), (1,)), ((), ())),
                         preferred_element_type=jnp.float32)
    E1 = (lax.broadcasted_iota(jnp.int32, (20, 4), 0) // 5
          == lax.broadcasted_iota(jnp.int32, (20, 4), 1)).astype(jnp.float32)
    E2 = (lax.broadcasted_iota(jnp.int32, (20, 5), 0) % 5
          == lax.broadcasted_iota(jnp.int32, (20, 5), 1)).astype(jnp.float32)
    pairsum = (jnp.dot(E1, ca, preferred_element_type=jnp.float32)
               + jnp.dot(E2, eb, preferred_element_type=jnp.float32)
               + c1_ref[...][None, :])
    h = jnp.maximum(pairsum, 0.0)
    logits = lax.dot_general(h, V2_ref[...], (((1,), (1,)), ((), ())),
                             preferred_element_type=jnp.float32)
    E3 = (lax.broadcasted_iota(jnp.int32, (20, 5), 0) % 5
          == lax.broadcasted_iota(jnp.int32, (20, 5), 1)).astype(jnp.float32)
    E4 = (lax.broadcasted_iota(jnp.int32, (4, 20), 0)
          == lax.broadcasted_iota(jnp.int32, (4, 20), 1) // 5).astype(jnp.float32)
    l45 = jnp.dot(E4, logits * E3, preferred_element_type=jnp.float32)
    cons = jax.nn.sigmoid(l45 + c2_ref[0])
    cons_ref[...] = cons
    conf_ref[0] = jnp.max(cons)


def kernel(claim_embedding, keys, values, Wq, bq, Wk, bk, V1, c1, V2, c2):
    scores = pl.pallas_call(
        _score_body,
        grid=(NBLK,),
        in_specs=[
            pl.BlockSpec((4, D), lambda i: (0, 0)),
            pl.BlockSpec((D, D), lambda i: (0, 0)),
            pl.BlockSpec((D,), lambda i: (0,)),
            pl.BlockSpec((D, D), lambda i: (0, 0)),
            pl.BlockSpec((D, 1), lambda i: (0, 0)),
            pl.BlockSpec((PB, D), lambda i: (i, 0)),
        ],
        out_specs=pl.BlockSpec((PB,), lambda i: (i,)),
        out_shape=jax.ShapeDtypeStruct((PADTOT,), jnp.float32),
    )(claim_embedding, Wq, bq, Wk, bk.reshape(D, 1), keys)

    mesh = plsc.VectorSubcoreMesh(core_axis_name="c", subcore_axis_name="s")
    cand_k, cand_i = pl.kernel(
        _sc_select,
        out_type=[
            jax.ShapeDtypeStruct((NW, 16), jnp.float32),
            jax.ShapeDtypeStruct((NW, 16), jnp.int32),
        ],
        mesh=mesh,
        scratch_types=[
            pltpu.VMEM((SLICE,), jnp.float32),
            pltpu.VMEM((16,), jnp.float32),
            pltpu.VMEM((16,), jnp.int32),
        ],
        compiler_params=pltpu.CompilerParams(needs_layout_passes=False),
    )(scores)

    retr, ts, cons, conf = pl.pallas_call(
        _select_mlp_body,
        in_specs=[
            pl.BlockSpec((NW, 16), lambda: (0, 0)),
            pl.BlockSpec((NW, 16), lambda: (0, 0)),
            pl.BlockSpec(memory_space=pl.ANY),
            pl.BlockSpec((4, D), lambda: (0, 0)),
            pl.BlockSpec((D, 2 * D), lambda: (0, 0)),
            pl.BlockSpec((D,), lambda: (0,)),
            pl.BlockSpec((1, D), lambda: (0, 0)),
            pl.BlockSpec(memory_space=pltpu.SMEM),
        ],
        out_specs=[
            pl.BlockSpec((5, D), lambda: (0, 0)),
            pl.BlockSpec(memory_space=pltpu.SMEM),
            pl.BlockSpec((4, 5), lambda: (0, 0)),
            pl.BlockSpec(memory_space=pltpu.SMEM),
        ],
        out_shape=[
            jax.ShapeDtypeStruct((5, D), jnp.float32),
            jax.ShapeDtypeStruct((5,), jnp.float32),
            jax.ShapeDtypeStruct((4, 5), jnp.float32),
            jax.ShapeDtypeStruct((1,), jnp.float32),
        ],
        scratch_shapes=[
            pltpu.VMEM((NW, 16), jnp.float32),
            pltpu.SemaphoreType.DMA,
        ],
    )(cand_k, cand_i, values, claim_embedding, V1, c1, V2, c2)

    return (retr, ts, cons, conf[0])


# speedup vs baseline: 1.1012x; 1.1012x over previous
import functools

import jax
import jax.numpy as jnp
from jax import lax
from jax.experimental import pallas as pl
from jax.experimental.pallas import tpu as pltpu
from jax.experimental.pallas import tpu_sc as plsc

D = 256
KEV = 100000
PB = 8192
NBLK = -(-KEV // PB)
PADTOT = NBLK * PB
NEG = float("-inf")

NW = 32
SLICE = PADTOT // NW
NCH = SLICE // 16


def _score_body(claim_ref, Wq_ref, bq_ref, Wk_ref, bkc_ref,
                keys_ref, out_ref):
    c0 = claim_ref[0:1, :]
    qr = lax.dot_general(c0, Wq_ref[...], (((1,), (1,)), ((), ())),
                         preferred_element_type=jnp.float32)
    qr = qr + bq_ref[...][None, :]
    qn = qr / jnp.maximum(jnp.sqrt(jnp.sum(qr * qr)), 1e-12)
    zT = lax.dot_general(Wk_ref[...], keys_ref[...], (((1,), (1,)), ((), ())),
                         preferred_element_type=jnp.float32)
    zT = zT + bkc_ref[...]
    num = lax.dot_general(qn, zT, (((1,), (0,)), ((), ())),
                          preferred_element_type=jnp.float32)
    d2 = jnp.sum(zT * zT, axis=0, keepdims=True)
    s = num / jnp.maximum(jnp.sqrt(d2), 1e-12)
    gid = (lax.broadcasted_iota(jnp.int32, (1, PB), 1)
           + pl.program_id(0) * PB)
    out_ref[...] = jnp.where(gid < KEV, s, NEG).reshape(PB)


def _merge16(bk, bi, ck, ci):
    ck, ci = plsc.sort_key_val(ck, ci, descending=True)
    bk, bi = plsc.sort_key_val(bk, bi)
    take = ck >= bk
    return jnp.where(take, ck, bk), jnp.where(take, ci, bi)


def _sc_select(scores_hbm, ck_out, ci_out, chunk_v, stage_k, stage_i):
    wid = lax.axis_index("c") * 16 + lax.axis_index("s")
    pltpu.sync_copy(scores_hbm.at[pl.ds(wid * SLICE, SLICE)], chunk_v)
    lanes = lax.iota(jnp.int32, 16)

    def body(j, carry):
        bk, bi = carry
        v = chunk_v[pl.ds(j * 16, 16)]
        gi = wid * SLICE + j * 16 + lanes
        return _merge16(bk, bi, v, gi)

    bk0 = jnp.full((16,), NEG, jnp.float32)
    bi0 = jnp.zeros((16,), jnp.int32)
    bk, bi = lax.fori_loop(0, NCH, body, (bk0, bi0))
    stage_k[...] = bk
    stage_i[...] = bi
    pltpu.sync_copy(stage_k, ck_out.at[wid])
    pltpu.sync_copy(stage_i, ci_out.at[wid])


def _select_mlp_body(ck_ref, ci_ref, values_ref, claim_ref, V1_ref, c1_ref,
                     V2_ref, c2_ref, retr_ref, ts_ref, cons_ref, conf_ref,
                     k_ref, sem):
    k_ref[...] = ck_ref[...]
    for i in range(5):
        k = k_ref[...]
        m = jnp.max(k)
        idx = jnp.min(jnp.where(k == m, ci_ref[...], KEV))
        ts_ref[i] = m
        cp = pltpu.make_async_copy(values_ref.at[pl.ds(idx, 1), :],
                                   retr_ref.at[pl.ds(i, 1), :], sem)
        cp.start()
        cp.wait()
        k_ref[...] = jnp.where(ci_ref[...] == idx, NEG, k)
    ev = retr_ref[...]
    A = V1_ref[:, 0:D]
    Bm = V1_ref[:, D:2 * D]
    ca = lax.dot_general(claim_ref[...], A, (((1,), (1,)), ((), ())),
                         preferred_element_type=jnp.float32)
    eb = lax.dot_general(ev, Bm, (((1,), (1,)), ((), ())),
                         preferred_element_type=jnp.float32)
    E1 = (lax.broadcasted_iota(jnp.int32, (20, 4), 0) // 5
          == lax.broadcasted_iota(jnp.int32, (20, 4), 1)).astype(jnp.float32)
    E2 = (lax.broadcasted_iota(jnp.int32, (20, 5), 0) % 5
          == lax.broadcasted_iota(jnp.int32, (20, 5), 1)).astype(jnp.float32)
    pairsum = (jnp.dot(E1, ca, preferred_element_type=jnp.float32)
               + jnp.dot(E2, eb, preferred_element_type=jnp.float32)
               + c1_ref[...][None, :])
    h = jnp.maximum(pairsum, 0.0)
    logits = lax.dot_general(h, V2_ref[...], (((1,), (1,)), ((), ())),
                             preferred_element_type=jnp.float32)
    E3 = (lax.broadcasted_iota(jnp.int32, (20, 5), 0) % 5
          == lax.broadcasted_iota(jnp.int32, (20, 5), 1)).astype(jnp.float32)
    E4 = (lax.broadcasted_iota(jnp.int32, (4, 20), 0)
          == lax.broadcasted_iota(jnp.int32, (4, 20), 1) // 5).astype(jnp.float32)
    l45 = jnp.dot(E4, logits * E3, preferred_element_type=jnp.float32)
    cons = jax.nn.sigmoid(l45 + c2_ref[0])
    cons_ref[...] = cons
    conf_ref[0] = jnp.max(cons)


def kernel(claim_embedding, keys, values, Wq, bq, Wk, bk, V1, c1, V2, c2):
    scores = pl.pallas_call(
        _score_body,
        grid=(NBLK,),
        in_specs=[
            pl.BlockSpec((4, D), lambda i: (0, 0)),
            pl.BlockSpec((D, D), lambda i: (0, 0)),
            pl.BlockSpec((D,), lambda i: (0,)),
            pl.BlockSpec((D, D), lambda i: (0, 0)),
            pl.BlockSpec((D, 1), lambda i: (0, 0)),
            pl.BlockSpec((PB, D), lambda i: (i, 0)),
        ],
        out_specs=pl.BlockSpec((PB,), lambda i: (i,)),
        out_shape=jax.ShapeDtypeStruct((PADTOT,), jnp.float32),
    )(claim_embedding, Wq, bq, Wk, bk.reshape(D, 1), keys)

    mesh = plsc.VectorSubcoreMesh(core_axis_name="c", subcore_axis_name="s")
    cand_k, cand_i = pl.kernel(
        _sc_select,
        out_type=[
            jax.ShapeDtypeStruct((NW, 16), jnp.float32),
            jax.ShapeDtypeStruct((NW, 16), jnp.int32),
        ],
        mesh=mesh,
        scratch_types=[
            pltpu.VMEM((SLICE,), jnp.float32),
            pltpu.VMEM((16,), jnp.float32),
            pltpu.VMEM((16,), jnp.int32),
        ],
        compiler_params=pltpu.CompilerParams(needs_layout_passes=False),
    )(scores)

    retr, ts, cons, conf = pl.pallas_call(
        _select_mlp_body,
        in_specs=[
            pl.BlockSpec((NW, 16), lambda: (0, 0)),
            pl.BlockSpec((NW, 16), lambda: (0, 0)),
            pl.BlockSpec(memory_space=pl.ANY),
            pl.BlockSpec((4, D), lambda: (0, 0)),
            pl.BlockSpec((D, 2 * D), lambda: (0, 0)),
            pl.BlockSpec((D,), lambda: (0,)),
            pl.BlockSpec((1, D), lambda: (0, 0)),
            pl.BlockSpec(memory_space=pltpu.SMEM),
        ],
        out_specs=[
            pl.BlockSpec((5, D), lambda: (0, 0)),
            pl.BlockSpec(memory_space=pltpu.SMEM),
            pl.BlockSpec((4, 5), lambda: (0, 0)),
            pl.BlockSpec(memory_space=pltpu.SMEM),
        ],
        out_shape=[
            jax.ShapeDtypeStruct((5, D), jnp.float32),
            jax.ShapeDtypeStruct((5,), jnp.float32),
            jax.ShapeDtypeStruct((4, 5), jnp.float32),
            jax.ShapeDtypeStruct((1,), jnp.float32),
        ],
        scratch_shapes=[
            pltpu.VMEM((NW, 16), jnp.float32),
            pltpu.SemaphoreType.DMA,
        ],
    )(cand_k, cand_i, values, claim_embedding, V1, c1, V2, c2)

    return (retr, ts, cons, conf[0])
